# baseline (speedup 1.0000x reference)
import jax
import jax.numpy as jnp
from jax import lax
from jax.experimental import pallas as pl
from jax.experimental.pallas import tpu as pltpu

_B, _C, _H, _W = 64, 768, 14, 14
_S = _H * _W
_E = 64
_PC = 7
_NSTEP = _S // _PC


def _body(x_ref, wt_ref, b_ref, o_ref, acc_ref):
    part = jnp.sum(x_ref[...], axis=0)

    @pl.when(pl.program_id(0) == 0)
    def _init():
        acc_ref[...] = part

    @pl.when(pl.program_id(0) > 0)
    def _accum():
        acc_ref[...] += part

    @pl.when(pl.program_id(0) == _NSTEP - 1)
    def _finish():
        pooled = acc_ref[...] * (1.0 / _S)
        logits = lax.dot_general(
            pooled, wt_ref[...], (((1,), (1,)), ((), ())),
            preferred_element_type=jnp.float32,
        ) + b_ref[...]
        mx = jnp.max(logits, axis=1, keepdims=True)
        e = jnp.exp(logits - mx)
        o_ref[...] = e / jnp.sum(e, axis=1, keepdims=True)


def kernel(x, W, b):
    xp = jnp.transpose(x, (2, 3, 0, 1)).reshape(_S, _B, _C)
    b2 = b.reshape(1, _E)
    return pl.pallas_call(
        _body,
        grid=(_NSTEP,),
        in_specs=[
            pl.BlockSpec((_PC, _B, _C), lambda i: (i, 0, 0)),
            pl.BlockSpec((_E, _C), lambda i: (0, 0)),
            pl.BlockSpec((1, _E), lambda i: (0, 0)),
        ],
        out_specs=pl.BlockSpec((_B, _E), lambda i: (0, 0)),
        out_shape=jax.ShapeDtypeStruct((_B, _E), jnp.float32),
        scratch_shapes=[pltpu.VMEM((_B, _C), jnp.float32)],
    )(xp, W, b2)


# speedup vs baseline: 1.7442x; 1.7442x over previous
import jax
import jax.numpy as jnp
from jax import lax
from jax.experimental import pallas as pl
from jax.experimental.pallas import tpu as pltpu

_B, _C, _H, _W = 64, 768, 14, 14
_S = _H * _W
_E = 64
_PC = 28
_NSTEP = _S // _PC


def _body(x_ref, wt_ref, b_ref, o_ref, acc_ref):
    part = jnp.sum(x_ref[...], axis=0)

    @pl.when(pl.program_id(0) == 0)
    def _init():
        acc_ref[...] = part

    @pl.when(pl.program_id(0) > 0)
    def _accum():
        acc_ref[...] += part

    @pl.when(pl.program_id(0) == _NSTEP - 1)
    def _finish():
        pooled = acc_ref[...] * (1.0 / _S)
        logits = lax.dot_general(
            pooled, wt_ref[...], (((1,), (1,)), ((), ())),
            preferred_element_type=jnp.float32,
        ) + b_ref[...]
        mx = jnp.max(logits, axis=1, keepdims=True)
        e = jnp.exp(logits - mx)
        o_ref[...] = e / jnp.sum(e, axis=1, keepdims=True)


def kernel(x, W, b):
    xp = jnp.transpose(x, (2, 3, 0, 1)).reshape(_S, _B, _C)
    b2 = b.reshape(1, _E)
    return pl.pallas_call(
        _body,
        grid=(_NSTEP,),
        in_specs=[
            pl.BlockSpec((_PC, _B, _C), lambda i: (i, 0, 0)),
            pl.BlockSpec((_E, _C), lambda i: (0, 0)),
            pl.BlockSpec((1, _E), lambda i: (0, 0)),
        ],
        out_specs=pl.BlockSpec((_B, _E), lambda i: (0, 0)),
        out_shape=jax.ShapeDtypeStruct((_B, _E), jnp.float32),
        scratch_shapes=[pltpu.VMEM((_B, _C), jnp.float32)],
    )(xp, W, b2)
